# baseline (speedup 1.0000x reference)
import functools

import jax
import jax.numpy as jnp
from jax import lax
from jax.experimental import pallas as pl
from jax.experimental.pallas import tpu as pltpu
from jax.experimental.pallas import tpu_sc as plsc

NC_ = 10000
NK_ = 2000
H = 256
HH = 128
A = 128
E = 160000
EP = 163840
PW = EP // 32
CH = 128
NCHUNK = EP // 16 // CH
NPAD = 10112
NKP = 2048
ROWS = 10240
RPT = ROWS // 16
DUMP = NC_

_f32 = jnp.float32


def _st_body(clus_ref, cell_ref, wcfc_ref, wclfc_ref, wattn_ref, s_ref, t_ref):
    w1 = wattn_ref[:, :A]
    w2 = wattn_ref[:, A:]
    u_s = lax.dot_general(w1, wclfc_ref[...], (((1,), (0,)), ((), ())))
    u_t = lax.dot_general(w2, wcfc_ref[...], (((1,), (0,)), ((), ())))
    s_ref[...] = lax.dot_general(clus_ref[...], u_s, (((1,), (1,)), ((), ())))
    t_ref[...] = lax.dot_general(cell_ref[...], u_t, (((1,), (1,)), ((), ())))


_st_call = pl.pallas_call(
    _st_body,
    out_shape=(
        jax.ShapeDtypeStruct((NK_, 1), _f32),
        jax.ShapeDtypeStruct((NC_, 1), _f32),
    ),
)


_sc_mesh = plsc.VectorSubcoreMesh(core_axis_name="c", subcore_axis_name="s")


@functools.partial(
    pl.kernel,
    out_type=(
        jax.ShapeDtypeStruct((EP,), _f32),
        jax.ShapeDtypeStruct((32, NPAD), _f32),
        jax.ShapeDtypeStruct((32, 4, NPAD), _f32),
    ),
    mesh=_sc_mesh,
    compiler_params=pltpu.CompilerParams(needs_layout_passes=False),
    scratch_types=(
        pltpu.VMEM((NKP,), _f32),
        pltpu.VMEM((NPAD,), _f32),
        pltpu.VMEM((PW,), jnp.int32),
        pltpu.VMEM((PW,), jnp.int32),
        pltpu.VMEM((4, PW), jnp.int32),
        pltpu.VMEM((PW,), _f32),
        pltpu.VMEM((NPAD,), _f32),
        pltpu.VMEM((NPAD,), _f32),
        pltpu.VMEM((NPAD,), _f32),
        pltpu.VMEM((NPAD,), _f32),
        pltpu.VMEM((NPAD,), _f32),
    ),
)
def _scalar_kernel(s_hbm, t_hbm, msrc_hbm, mdst_hbm, d0_hbm, d1_hbm, d2_hbm,
                   d3_hbm, ex_out, den_out, cnt_out,
                   s_tab, t_tab, msrc_v, mdst_v, dst_v, ex_v, den_v,
                   cnt0_v, cnt1_v, cnt2_v, cnt3_v):
    cnt_refs = (cnt0_v, cnt1_v, cnt2_v, cnt3_v)
    w = lax.axis_index("s") * 2 + lax.axis_index("c")
    base_e = w * PW
    pltpu.sync_copy(s_hbm, s_tab)
    pltpu.sync_copy(t_hbm, t_tab)
    pltpu.sync_copy(msrc_hbm.at[pl.ds(base_e, PW)], msrc_v)
    pltpu.sync_copy(mdst_hbm.at[pl.ds(base_e, PW)], mdst_v)
    pltpu.sync_copy(d0_hbm.at[pl.ds(base_e, PW)], dst_v.at[0])
    pltpu.sync_copy(d1_hbm.at[pl.ds(base_e, PW)], dst_v.at[1])
    pltpu.sync_copy(d2_hbm.at[pl.ds(base_e, PW)], dst_v.at[2])
    pltpu.sync_copy(d3_hbm.at[pl.ds(base_e, PW)], dst_v.at[3])

    zeros = jnp.zeros((16,), _f32)

    def zero_step(i, c):
        den_v[pl.ds(i * 16, 16)] = zeros
        for k in range(4):
            cnt_refs[k][pl.ds(i * 16, 16)] = zeros
        return c

    lax.fori_loop(0, NPAD // 16, zero_step, 0)

    ones = jnp.ones((16,), _f32)

    def step(i, c):
        base = i * 16
        msv = msrc_v[pl.ds(base, 16)]
        mdv = mdst_v[pl.ds(base, 16)]
        sv = plsc.load_gather(s_tab, [msv])
        tv = plsc.load_gather(t_tab, [mdv])
        x = sv + tv
        aa = jnp.where(x > 0, x, x * _f32(0.01))
        ex = jnp.exp(aa)
        ex_v[pl.ds(base, 16)] = ex
        plsc.addupdate_scatter(den_v, [mdv], ex)
        for k in range(4):
            dv = dst_v[k, pl.ds(base, 16)]
            plsc.addupdate_scatter(cnt_refs[k], [dv], ones)
        return c

    lax.fori_loop(0, PW // 16, step, 0)

    pltpu.sync_copy(ex_v, ex_out.at[pl.ds(base_e, PW)])
    pltpu.sync_copy(den_v, den_out.at[w])
    for k in range(4):
        pltpu.sync_copy(cnt_refs[k], cnt_out.at[w, k])


def _prep_body(denp_ref, cntp_ref, wd_ref, wl_ref, wg_ref, wc_ref, wm_ref,
               wu_ref, invden_ref, invc_ref, flags_ref, p_ref, q_ref, u0_ref):
    den = jnp.sum(denp_ref[...], axis=0)
    cnt = jnp.sum(cntp_ref[...], axis=0)
    invden_ref[...] = (1.0 / jnp.maximum(den, 1e-9))[None, :]
    invc_ref[...] = (1.0 / jnp.maximum(cnt, 1.0)).T
    f4 = (cnt > 0).astype(_f32)
    fm = (den > 0).astype(_f32)[None, :]
    flags_ref[...] = jnp.concatenate([f4, fm], axis=0).T
    u0_ref[...] = wu_ref[:, :H]
    for e, wref in enumerate((wd_ref, wl_ref, wg_ref, wc_ref, wm_ref)):
        u = wu_ref[:, (e + 1) * H:(e + 2) * H]
        p_ref[e] = lax.dot_general(u, wref[:, :H], (((1,), (0,)), ((), ())))
        q_ref[e] = lax.dot_general(u, wref[:, H:], (((1,), (0,)), ((), ())))


_prep_call = pl.pallas_call(
    _prep_body,
    out_shape=(
        jax.ShapeDtypeStruct((1, NPAD), _f32),
        jax.ShapeDtypeStruct((NPAD, 4), _f32),
        jax.ShapeDtypeStruct((NPAD, 5), _f32),
        jax.ShapeDtypeStruct((5, H, H), _f32),
        jax.ShapeDtypeStruct((5, H, H), _f32),
        jax.ShapeDtypeStruct((H, H), _f32),
    ),
)


@functools.partial(
    pl.kernel,
    out_type=jax.ShapeDtypeStruct((2, 5, ROWS, HH), _f32),
    mesh=_sc_mesh,
    compiler_params=pltpu.CompilerParams(needs_layout_passes=False),
    scratch_types=(
        pltpu.VMEM_SHARED((ROWS, HH), _f32),
        pltpu.VMEM((CH, HH), _f32),
        pltpu.VMEM((CH, HH), _f32),
        pltpu.VMEM((NCHUNK // 5, CH), jnp.int32),
        pltpu.VMEM((NCHUNK // 5, CH), jnp.int32),
        pltpu.VMEM((CH,), _f32),
        pltpu.VMEM((NPAD,), _f32),
        pltpu.SemaphoreType.DMA,
        pltpu.SemaphoreType.DMA,
        pltpu.SemaphoreType.DMA,
        pltpu.SemaphoreType.DMA,
    ),
)
def _feat_kernel(cl_l, cl_r, cu_l, cu_r, src3, dst3, ex3, invd_hbm, acc_out,
                 acc_sh, gbuf, gbuf1, srcb, dstb, alphab, invd_v,
                 sem, sem1, ssem, ssem1):
    h = lax.axis_index("c")
    t = lax.axis_index("s")
    rbase = t * RPT
    nh = NCHUNK // 5
    pltpu.sync_copy(invd_hbm, invd_v)

    zeros = jnp.zeros((16,), _f32)

    for e in range(5):
        tbl_l = (cl_l, cl_l, cl_l, cl_l, cu_l)[e]
        tbl_r = (cl_r, cl_r, cl_r, cl_r, cu_r)[e]

        def zb_step(i, c):
            r = i // (HH // 16)
            cc = i % (HH // 16)
            gbuf[r, pl.ds(cc * 16, 16)] = zeros
            return c

        lax.fori_loop(0, CH * HH // 16, zb_step, 0)
        for j in range(RPT // CH):
            pltpu.sync_copy(gbuf, acc_sh.at[pl.ds(rbase + j * CH, CH)])
        plsc.subcore_barrier()

        bufs = (gbuf, gbuf1)
        sems = (sem, sem1)
        ssems = (ssem, ssem1)

        def start_gather(jj, b):
            idxr = srcb.at[jj]

            @pl.when(h == 0)
            def _g0():
                pltpu.async_copy(tbl_l.at[idxr], bufs[b], sems[b])

            @pl.when(h == 1)
            def _g1():
                pltpu.async_copy(tbl_r.at[idxr], bufs[b], sems[b])

        def wait_gather(b):
            pltpu.make_async_copy(tbl_l.at[srcb.at[0]], bufs[b],
                                  sems[b]).wait()

        def process(jj, jg, b):
            buf = bufs[b]
            if e == 4:
                pltpu.sync_copy(ex3.at[t, jg], alphab)
                for v in range(CH // 16):
                    dv = dstb[jj, pl.ds(v * 16, 16)]
                    ev = alphab[pl.ds(v * 16, 16)]
                    iv = plsc.load_gather(invd_v, [dv])
                    alphab[pl.ds(v * 16, 16)] = ev * iv

                def scale(g, c2):
                    av16 = alphab[pl.ds(g * 16, 16)]
                    for l in range(16):
                        r = g * 16 + l
                        av = av16[l]
                        for cc in range(HH // 16):
                            buf[r, pl.ds(cc * 16, 16)] = (
                                buf[r, pl.ds(cc * 16, 16)] * av)
                    return c2

                lax.fori_loop(0, CH // 16, scale, 0)
            pltpu.async_copy(buf, acc_sh.at[dstb.at[jj]], ssems[b], add=True)

        def wait_scatter(b):
            pltpu.make_async_copy(bufs[b], acc_sh.at[dstb.at[0]],
                                  ssems[b]).wait()

        for h2 in range(5):
            pltpu.sync_copy(src3.at[e, t, pl.ds(h2 * nh, nh)], srcb)
            pltpu.sync_copy(dst3.at[e, t, pl.ds(h2 * nh, nh)], dstb)

            start_gather(0, 0)
            start_gather(1, 1)

            def pair(j, c):
                wait_gather(0)
                process(j, h2 * nh + j, 0)
                wait_gather(1)
                process(j + 1, h2 * nh + j + 1, 1)
                wait_scatter(0)

                @pl.when(j + 2 < nh)
                def _nx0():
                    start_gather(j + 2, 0)

                wait_scatter(1)

                @pl.when(j + 3 < nh)
                def _nx1():
                    start_gather(j + 3, 1)

                return c

            lax.fori_loop(0, nh // 2, lambda i, c: pair(i * 2, c), 0)
        plsc.subcore_barrier()
        for j in range(RPT // CH):
            pltpu.sync_copy(
                acc_sh.at[pl.ds(rbase + j * CH, CH)],
                acc_out.at[h, e, pl.ds(rbase + j * CH, CH)])
        plsc.subcore_barrier()


_BD = 1000


def _upd_body(cell_ref, acc_ref, invc_ref, flags_ref, u0_ref, p_ref, q_ref,
              out_ref):
    hb = cell_ref[...]
    pre = lax.dot_general(hb, u0_ref[...], (((1,), (1,)), ((), ())))
    for e in range(5):
        al = acc_ref[0, e]
        ar = acc_ref[1, e]
        if e < 4:
            ic = invc_ref[:, e][:, None]
            al = al * ic
            ar = ar * ic
        pe = p_ref[e]
        pre += lax.dot_general(al, pe[:, :HH], (((1,), (1,)), ((), ())))
        pre += lax.dot_general(ar, pe[:, HH:], (((1,), (1,)), ((), ())))
        f = flags_ref[:, e][:, None]
        pre += f * lax.dot_general(hb, q_ref[e], (((1,), (1,)), ((), ())))
    out_ref[...] = jnp.maximum(pre, 0.0)


_upd_call = pl.pallas_call(
    _upd_body,
    grid=(NC_ // _BD,),
    in_specs=[
        pl.BlockSpec((_BD, H), lambda i: (i, 0)),
        pl.BlockSpec((2, 5, _BD, HH), lambda i: (0, 0, i, 0)),
        pl.BlockSpec((_BD, 4), lambda i: (i, 0)),
        pl.BlockSpec((_BD, 5), lambda i: (i, 0)),
        pl.BlockSpec((H, H), lambda i: (0, 0)),
        pl.BlockSpec((5, H, H), lambda i: (0, 0, 0)),
        pl.BlockSpec((5, H, H), lambda i: (0, 0, 0)),
    ],
    out_specs=pl.BlockSpec((_BD, H), lambda i: (i, 0)),
    out_shape=jax.ShapeDtypeStruct((NC_, H), _f32),
)


def kernel(cell_h, cluster_h, edge_diff, edge_lt, edge_gt, edge_contains,
           mc_src, mc_dst, W_diff, W_lt, W_gt, W_contains, W_may_contain,
           W_cell_fc, W_cluster_fc, W_attn, W_upd):
    s2, t2 = _st_call(cluster_h, cell_h, W_cell_fc, W_cluster_fc, W_attn)
    s_pad = jnp.concatenate([s2[:, 0], jnp.zeros((NKP - NK_,), _f32)])
    t_pad = jnp.concatenate([t2[:, 0], jnp.zeros((NPAD - NC_,), _f32)])

    def pad_i(x, fill):
        return jnp.concatenate(
            [x.astype(jnp.int32), jnp.full((EP - E,), fill, jnp.int32)])

    srcs = [pad_i(edge_diff[0], 0), pad_i(edge_lt[0], 0),
            pad_i(edge_gt[0], 0), pad_i(edge_contains[0], 0),
            pad_i(mc_src, 0)]
    dsts = [pad_i(edge_diff[1], DUMP), pad_i(edge_lt[1], DUMP),
            pad_i(edge_gt[1], DUMP), pad_i(edge_contains[1], DUMP),
            pad_i(mc_dst, DUMP)]

    ex, den_p, cnt_p = _scalar_kernel(
        s_pad, t_pad, srcs[4], dsts[4], dsts[0], dsts[1], dsts[2], dsts[3])

    invden2, invc, flags, P, Q, U0 = _prep_call(
        den_p, cnt_p, W_diff, W_lt, W_gt, W_contains, W_may_contain, W_upd)

    src3 = jnp.stack([x.reshape(16, NCHUNK, CH) for x in srcs])
    dst3 = jnp.stack([x.reshape(16, NCHUNK, CH) for x in dsts])
    ex3 = ex.reshape(16, NCHUNK, CH)

    acc = _feat_kernel(
        cell_h[:, :HH], cell_h[:, HH:], cluster_h[:, :HH], cluster_h[:, HH:],
        src3, dst3, ex3, invden2.reshape(NPAD))

    return _upd_call(cell_h, acc, invc, flags, U0, P, Q)


# speedup vs baseline: 1.0689x; 1.0689x over previous
import functools

import jax
import jax.numpy as jnp
from jax import lax
from jax.experimental import pallas as pl
from jax.experimental.pallas import tpu as pltpu
from jax.experimental.pallas import tpu_sc as plsc

NC_ = 10000
NK_ = 2000
H = 256
HH = 128
A = 128
E = 160000
EP = 163840
PW = EP // 32
CH = 128
NCHUNK = EP // 16 // CH
NPAD = 10112
NKP = 2048
ROWS = 10240
RPT = ROWS // 16
DUMP = NC_

_f32 = jnp.float32


def _st_body(clus_ref, cell_ref, wcfc_ref, wclfc_ref, wattn_ref, s_ref, t_ref):
    w1 = wattn_ref[:, :A]
    w2 = wattn_ref[:, A:]
    u_s = lax.dot_general(w1, wclfc_ref[...], (((1,), (0,)), ((), ())))
    u_t = lax.dot_general(w2, wcfc_ref[...], (((1,), (0,)), ((), ())))
    s_ref[...] = lax.dot_general(clus_ref[...], u_s, (((1,), (1,)), ((), ())))
    t_ref[...] = lax.dot_general(cell_ref[...], u_t, (((1,), (1,)), ((), ())))


_st_call = pl.pallas_call(
    _st_body,
    out_shape=(
        jax.ShapeDtypeStruct((NK_, 1), _f32),
        jax.ShapeDtypeStruct((NC_, 1), _f32),
    ),
)


_sc_mesh = plsc.VectorSubcoreMesh(core_axis_name="c", subcore_axis_name="s")


@functools.partial(
    pl.kernel,
    out_type=(
        jax.ShapeDtypeStruct((EP,), _f32),
        jax.ShapeDtypeStruct((32, NPAD), _f32),
        jax.ShapeDtypeStruct((32, 4, NPAD), _f32),
    ),
    mesh=_sc_mesh,
    compiler_params=pltpu.CompilerParams(needs_layout_passes=False),
    scratch_types=(
        pltpu.VMEM((NKP,), _f32),
        pltpu.VMEM((NPAD,), _f32),
        pltpu.VMEM((PW,), jnp.int32),
        pltpu.VMEM((PW,), jnp.int32),
        pltpu.VMEM((4, PW), jnp.int32),
        pltpu.VMEM((PW,), _f32),
        pltpu.VMEM((NPAD,), _f32),
        pltpu.VMEM((NPAD,), _f32),
        pltpu.VMEM((NPAD,), _f32),
        pltpu.VMEM((NPAD,), _f32),
        pltpu.VMEM((NPAD,), _f32),
    ),
)
def _scalar_kernel(s_hbm, t_hbm, msrc_hbm, mdst_hbm, d0_hbm, d1_hbm, d2_hbm,
                   d3_hbm, ex_out, den_out, cnt_out,
                   s_tab, t_tab, msrc_v, mdst_v, dst_v, ex_v, den_v,
                   cnt0_v, cnt1_v, cnt2_v, cnt3_v):
    cnt_refs = (cnt0_v, cnt1_v, cnt2_v, cnt3_v)
    w = lax.axis_index("s") * 2 + lax.axis_index("c")
    base_e = w * PW
    pltpu.sync_copy(s_hbm, s_tab)
    pltpu.sync_copy(t_hbm, t_tab)
    pltpu.sync_copy(msrc_hbm.at[pl.ds(base_e, PW)], msrc_v)
    pltpu.sync_copy(mdst_hbm.at[pl.ds(base_e, PW)], mdst_v)
    pltpu.sync_copy(d0_hbm.at[pl.ds(base_e, PW)], dst_v.at[0])
    pltpu.sync_copy(d1_hbm.at[pl.ds(base_e, PW)], dst_v.at[1])
    pltpu.sync_copy(d2_hbm.at[pl.ds(base_e, PW)], dst_v.at[2])
    pltpu.sync_copy(d3_hbm.at[pl.ds(base_e, PW)], dst_v.at[3])

    zeros = jnp.zeros((16,), _f32)

    def zero_step(i, c):
        den_v[pl.ds(i * 16, 16)] = zeros
        for k in range(4):
            cnt_refs[k][pl.ds(i * 16, 16)] = zeros
        return c

    lax.fori_loop(0, NPAD // 16, zero_step, 0)

    ones = jnp.ones((16,), _f32)

    def step(i, c):
        base = i * 16
        msv = msrc_v[pl.ds(base, 16)]
        mdv = mdst_v[pl.ds(base, 16)]
        sv = plsc.load_gather(s_tab, [msv])
        tv = plsc.load_gather(t_tab, [mdv])
        x = sv + tv
        aa = jnp.where(x > 0, x, x * _f32(0.01))
        ex = jnp.exp(aa)
        ex_v[pl.ds(base, 16)] = ex
        plsc.addupdate_scatter(den_v, [mdv], ex)
        for k in range(4):
            dv = dst_v[k, pl.ds(base, 16)]
            plsc.addupdate_scatter(cnt_refs[k], [dv], ones)
        return c

    lax.fori_loop(0, PW // 16, step, 0)

    pltpu.sync_copy(ex_v, ex_out.at[pl.ds(base_e, PW)])
    pltpu.sync_copy(den_v, den_out.at[w])
    for k in range(4):
        pltpu.sync_copy(cnt_refs[k], cnt_out.at[w, k])


def _prep_body(denp_ref, cntp_ref, wd_ref, wl_ref, wg_ref, wc_ref, wm_ref,
               wu_ref, invden_ref, invc_ref, flags_ref, p_ref, q_ref, u0_ref):
    den = jnp.sum(denp_ref[...], axis=0)
    cnt = jnp.sum(cntp_ref[...], axis=0)
    invden_ref[...] = (1.0 / jnp.maximum(den, 1e-9))[None, :]
    invc_ref[...] = (1.0 / jnp.maximum(cnt, 1.0)).T
    f4 = (cnt > 0).astype(_f32)
    fm = (den > 0).astype(_f32)[None, :]
    flags_ref[...] = jnp.concatenate([f4, fm], axis=0).T
    u0_ref[...] = wu_ref[:, :H]
    for e, wref in enumerate((wd_ref, wl_ref, wg_ref, wc_ref, wm_ref)):
        u = wu_ref[:, (e + 1) * H:(e + 2) * H]
        p_ref[e] = lax.dot_general(u, wref[:, :H], (((1,), (0,)), ((), ())))
        q_ref[e] = lax.dot_general(u, wref[:, H:], (((1,), (0,)), ((), ())))


_prep_call = pl.pallas_call(
    _prep_body,
    out_shape=(
        jax.ShapeDtypeStruct((1, NPAD), _f32),
        jax.ShapeDtypeStruct((NPAD, 4), _f32),
        jax.ShapeDtypeStruct((NPAD, 5), _f32),
        jax.ShapeDtypeStruct((5, H, H), _f32),
        jax.ShapeDtypeStruct((5, H, H), _f32),
        jax.ShapeDtypeStruct((H, H), _f32),
    ),
)


@functools.partial(
    pl.kernel,
    out_type=jax.ShapeDtypeStruct((2, 5, ROWS, HH), _f32),
    mesh=_sc_mesh,
    compiler_params=pltpu.CompilerParams(needs_layout_passes=False),
    scratch_types=(
        pltpu.VMEM_SHARED((ROWS, HH), _f32),
        pltpu.VMEM((CH, HH), _f32),
        pltpu.VMEM((CH, HH), _f32),
        pltpu.VMEM((NCHUNK // 5, CH), jnp.int32),
        pltpu.VMEM((NCHUNK // 5, CH), jnp.int32),
        pltpu.VMEM((CH,), _f32),
        pltpu.VMEM((NPAD,), _f32),
        pltpu.SemaphoreType.DMA,
        pltpu.SemaphoreType.DMA,
        pltpu.SemaphoreType.DMA,
        pltpu.SemaphoreType.DMA,
    ),
)
def _feat_kernel(cl_l, cl_r, cu_l, cu_r, src3, dst3, ex3, invd_hbm, acc_out,
                 acc_sh, gbuf, gbuf1, srcb, dstb, alphab, invd_v,
                 sem, sem1, ssem, ssem1):
    h = lax.axis_index("c")
    t = lax.axis_index("s")
    rbase = t * RPT
    nh = NCHUNK // 5
    pltpu.sync_copy(invd_hbm, invd_v)

    zeros = jnp.zeros((16,), _f32)

    for e in range(5):
        tbl_l = (cl_l, cl_l, cl_l, cl_l, cu_l)[e]
        tbl_r = (cl_r, cl_r, cl_r, cl_r, cu_r)[e]

        def zb_step(i, c):
            r = i // (HH // 16)
            cc = i % (HH // 16)
            gbuf[r, pl.ds(cc * 16, 16)] = zeros
            return c

        lax.fori_loop(0, CH * HH // 16, zb_step, 0)
        for j in range(RPT // CH):
            pltpu.sync_copy(gbuf, acc_sh.at[pl.ds(rbase + j * CH, CH)])
        plsc.subcore_barrier()

        bufs = (gbuf, gbuf1)
        sems = (sem, sem1)
        ssems = (ssem, ssem1)

        def start_gather(jj, b):
            idxr = srcb.at[jj]

            @pl.when(h == 0)
            def _g0():
                pltpu.async_copy(tbl_l.at[idxr], bufs[b], sems[b])

            @pl.when(h == 1)
            def _g1():
                pltpu.async_copy(tbl_r.at[idxr], bufs[b], sems[b])

        def wait_gather(b):
            pltpu.make_async_copy(tbl_l.at[srcb.at[0]], bufs[b],
                                  sems[b]).wait()

        def process(jj, jg, b):
            buf = bufs[b]
            if e == 4:
                pltpu.sync_copy(ex3.at[t, jg], alphab)
                for v in range(CH // 16):
                    dv = dstb[jj, pl.ds(v * 16, 16)]
                    ev = alphab[pl.ds(v * 16, 16)]
                    iv = plsc.load_gather(invd_v, [dv])
                    alphab[pl.ds(v * 16, 16)] = ev * iv

                def scale(g, c2):
                    av16 = alphab[pl.ds(g * 16, 16)]
                    for l in range(16):
                        r = g * 16 + l
                        av = av16[l]
                        for cc in range(HH // 16):
                            buf[r, pl.ds(cc * 16, 16)] = (
                                buf[r, pl.ds(cc * 16, 16)] * av)
                    return c2

                lax.fori_loop(0, CH // 16, scale, 0)
            pltpu.sync_copy(buf, acc_sh.at[dstb.at[jj]], add=True)

        for h2 in range(5):
            pltpu.sync_copy(src3.at[e, t, pl.ds(h2 * nh, nh)], srcb)
            pltpu.sync_copy(dst3.at[e, t, pl.ds(h2 * nh, nh)], dstb)

            start_gather(0, 0)

            def pair(j, c):
                start_gather(j + 1, 1)
                wait_gather(0)
                process(j, h2 * nh + j, 0)

                @pl.when(j + 2 < nh)
                def _nx0():
                    start_gather(j + 2, 0)

                wait_gather(1)
                process(j + 1, h2 * nh + j + 1, 1)
                return c

            lax.fori_loop(0, nh // 2, lambda i, c: pair(i * 2, c), 0)
        plsc.subcore_barrier()
        for j in range(RPT // CH):
            pltpu.sync_copy(
                acc_sh.at[pl.ds(rbase + j * CH, CH)],
                acc_out.at[h, e, pl.ds(rbase + j * CH, CH)])
        plsc.subcore_barrier()


_BD = 1000


def _upd_body(cell_ref, acc_ref, invc_ref, flags_ref, u0_ref, p_ref, q_ref,
              out_ref):
    hb = cell_ref[...]
    pre = lax.dot_general(hb, u0_ref[...], (((1,), (1,)), ((), ())))
    for e in range(5):
        al = acc_ref[0, e]
        ar = acc_ref[1, e]
        if e < 4:
            ic = invc_ref[:, e][:, None]
            al = al * ic
            ar = ar * ic
        pe = p_ref[e]
        pre += lax.dot_general(al, pe[:, :HH], (((1,), (1,)), ((), ())))
        pre += lax.dot_general(ar, pe[:, HH:], (((1,), (1,)), ((), ())))
        f = flags_ref[:, e][:, None]
        pre += f * lax.dot_general(hb, q_ref[e], (((1,), (1,)), ((), ())))
    out_ref[...] = jnp.maximum(pre, 0.0)


_upd_call = pl.pallas_call(
    _upd_body,
    grid=(NC_ // _BD,),
    in_specs=[
        pl.BlockSpec((_BD, H), lambda i: (i, 0)),
        pl.BlockSpec((2, 5, _BD, HH), lambda i: (0, 0, i, 0)),
        pl.BlockSpec((_BD, 4), lambda i: (i, 0)),
        pl.BlockSpec((_BD, 5), lambda i: (i, 0)),
        pl.BlockSpec((H, H), lambda i: (0, 0)),
        pl.BlockSpec((5, H, H), lambda i: (0, 0, 0)),
        pl.BlockSpec((5, H, H), lambda i: (0, 0, 0)),
    ],
    out_specs=pl.BlockSpec((_BD, H), lambda i: (i, 0)),
    out_shape=jax.ShapeDtypeStruct((NC_, H), _f32),
)


def kernel(cell_h, cluster_h, edge_diff, edge_lt, edge_gt, edge_contains,
           mc_src, mc_dst, W_diff, W_lt, W_gt, W_contains, W_may_contain,
           W_cell_fc, W_cluster_fc, W_attn, W_upd):
    s2, t2 = _st_call(cluster_h, cell_h, W_cell_fc, W_cluster_fc, W_attn)
    s_pad = jnp.concatenate([s2[:, 0], jnp.zeros((NKP - NK_,), _f32)])
    t_pad = jnp.concatenate([t2[:, 0], jnp.zeros((NPAD - NC_,), _f32)])

    def pad_i(x, fill):
        return jnp.concatenate(
            [x.astype(jnp.int32), jnp.full((EP - E,), fill, jnp.int32)])

    srcs = [pad_i(edge_diff[0], 0), pad_i(edge_lt[0], 0),
            pad_i(edge_gt[0], 0), pad_i(edge_contains[0], 0),
            pad_i(mc_src, 0)]
    dsts = [pad_i(edge_diff[1], DUMP), pad_i(edge_lt[1], DUMP),
            pad_i(edge_gt[1], DUMP), pad_i(edge_contains[1], DUMP),
            pad_i(mc_dst, DUMP)]

    ex, den_p, cnt_p = _scalar_kernel(
        s_pad, t_pad, srcs[4], dsts[4], dsts[0], dsts[1], dsts[2], dsts[3])

    invden2, invc, flags, P, Q, U0 = _prep_call(
        den_p, cnt_p, W_diff, W_lt, W_gt, W_contains, W_may_contain, W_upd)

    src3 = jnp.stack([x.reshape(16, NCHUNK, CH) for x in srcs])
    dst3 = jnp.stack([x.reshape(16, NCHUNK, CH) for x in dsts])
    ex3 = ex.reshape(16, NCHUNK, CH)

    acc = _feat_kernel(
        cell_h[:, :HH], cell_h[:, HH:], cluster_h[:, :HH], cluster_h[:, HH:],
        src3, dst3, ex3, invden2.reshape(NPAD))

    return _upd_call(cell_h, acc, invc, flags, U0, P, Q)
